# baseline (speedup 1.0000x reference)
import jax
import jax.numpy as jnp
from jax.experimental import pallas as pl

_N = 4096
_TILE = 512
_K = _N // 2
_WEIGHT = 3.0


def _topk_stats(x, n, k):
    mean_all = jnp.sum(x) / n
    xi = jax.lax.bitcast_convert_type(x, jnp.int32)

    def bit_body(b, kth):
        cand = kth | (jnp.int32(1) << (30 - b))
        cnt = jnp.sum(jnp.where(xi >= cand, jnp.int32(1), jnp.int32(0)))
        return jnp.where(cnt >= k, cand, kth)

    kth = jax.lax.fori_loop(0, 31, bit_body, jnp.int32(0))
    thr = jax.lax.bitcast_convert_type(kth, jnp.float32)
    gt_mask = xi > kth
    cnt_gt = jnp.sum(jnp.where(gt_mask, jnp.int32(1), jnp.int32(0)))
    sum_top = jnp.sum(jnp.where(gt_mask, x, 0.0)) + (k - cnt_gt).astype(jnp.float32) * thr
    return mean_all + _WEIGHT * sum_top / k


_DN = (((1,), (0,)), ((), ()))


def _dot(a, b):
    return jax.lax.dot_general(a, b, _DN, precision=jax.lax.Precision.HIGHEST,
                               preferred_element_type=jnp.float32)


def _chamfer_body(aA_ref, bA_ref, aB_ref, bB_ref, p2_ref, g2_ref, out_ref):
    bA = bA_ref[0]
    bB = bB_ref[0]

    def tile_body(t, carry):
        rmA, rmB = carry
        base = t * _TILE
        eA = _dot(aA_ref[0, pl.ds(base, _TILE), :], bA)
        rmA = jnp.minimum(rmA, jnp.min(eA, axis=0, keepdims=True))
        eB = _dot(aB_ref[0, pl.ds(base, _TILE), :], bB)
        rmB = jnp.minimum(rmB, jnp.min(eB, axis=0, keepdims=True))
        return rmA, rmB

    init = jnp.full((1, _N), jnp.inf, jnp.float32)
    rmA, rmB = jax.lax.fori_loop(0, _N // _TILE, tile_body, (init, init))
    d1 = jnp.sqrt(jnp.maximum(rmA + p2_ref[0], 0.0))
    d2 = jnp.sqrt(jnp.maximum(rmB + g2_ref[0], 0.0))
    loss = _topk_stats(d1, _N, _K) + _topk_stats(d2, _N, _K)
    out_ref[0] = jnp.full((8, 128), loss, jnp.float32)


def _round_bf16(x):
    i = jax.lax.bitcast_convert_type(x, jnp.uint32)
    i = (i + jnp.uint32(0x7FFF) + ((i >> 16) & jnp.uint32(1))) & jnp.uint32(0xFFFF0000)
    return jax.lax.bitcast_convert_type(i, jnp.float32)


def _augment(x):
    xb = _round_bf16(x)
    n2 = jnp.sum(x * x, axis=-1, keepdims=True)
    h1 = _round_bf16(n2)
    r1 = n2 - h1
    h2 = _round_bf16(r1)
    h3 = _round_bf16(r1 - h2)
    zero = jnp.zeros_like(n2)
    rows = jnp.concatenate([-2.0 * xb, h1, h2, h3, zero, zero], axis=-1)
    return rows, n2


def _cols(x):
    xt = _round_bf16(jnp.transpose(x, (0, 2, 1)))
    b, _, n = xt.shape
    ones = jnp.ones((b, 3, n), jnp.float32)
    zeros = jnp.zeros((b, 2, n), jnp.float32)
    return jnp.concatenate([xt, ones, zeros], axis=1)


@jax.jit
def kernel(pred_pointclouds, gt_pointclouds):
    b = pred_pointclouds.shape[0]
    aA, g2 = _augment(gt_pointclouds)
    aB, p2 = _augment(pred_pointclouds)
    bA = _cols(pred_pointclouds)
    bB = _cols(gt_pointclouds)
    p2r = jnp.transpose(p2, (0, 2, 1))
    g2r = jnp.transpose(g2, (0, 2, 1))
    out = pl.pallas_call(
        _chamfer_body,
        grid=(b,),
        in_specs=[
            pl.BlockSpec((1, _N, 8), lambda i: (i, 0, 0)),
            pl.BlockSpec((1, 8, _N), lambda i: (i, 0, 0)),
            pl.BlockSpec((1, _N, 8), lambda i: (i, 0, 0)),
            pl.BlockSpec((1, 8, _N), lambda i: (i, 0, 0)),
            pl.BlockSpec((1, 1, _N), lambda i: (i, 0, 0)),
            pl.BlockSpec((1, 1, _N), lambda i: (i, 0, 0)),
        ],
        out_specs=pl.BlockSpec((1, 8, 128), lambda i: (i, 0, 0)),
        out_shape=jax.ShapeDtypeStruct((b, 8, 128), jnp.float32),
    )(aA, bA, aB, bB, p2r, g2r)
    return jnp.sum(out[:, 0, 0]) / b


# speedup vs baseline: 1.5724x; 1.5724x over previous
import jax
import jax.numpy as jnp
from jax.experimental import pallas as pl
from jax.experimental.pallas import tpu as pltpu

_N = 4096
_TILE = 512
_K = _N // 2
_WEIGHT = 3.0


def _round_bf16(x):
    i = jax.lax.bitcast_convert_type(x, jnp.uint32)
    i = (i + jnp.uint32(0x7FFF) + ((i >> 16) & jnp.uint32(1))) & jnp.uint32(0xFFFF0000)
    return jax.lax.bitcast_convert_type(i, jnp.float32)


def _topk_stats(x, n, k):
    mean_all = jnp.sum(x) / n
    xi = jax.lax.bitcast_convert_type(x, jnp.int32)

    def bit_body(b, kth):
        cand = kth | (jnp.int32(1) << (30 - b))
        cnt = jnp.sum(jnp.where(xi >= cand, jnp.int32(1), jnp.int32(0)))
        return jnp.where(cnt >= k, cand, kth)

    kth = jax.lax.fori_loop(0, 31, bit_body, jnp.int32(0))
    thr = jax.lax.bitcast_convert_type(kth, jnp.float32)
    gt_mask = xi > kth
    cnt_gt = jnp.sum(jnp.where(gt_mask, jnp.int32(1), jnp.int32(0)))
    sum_top = jnp.sum(jnp.where(gt_mask, x, 0.0)) + (k - cnt_gt).astype(jnp.float32) * thr
    return mean_all + _WEIGHT * sum_top / k


def _chamfer_body(predT_ref, p2_ref, gts_ref, g2_ref, out_ref, s_ref):
    px = predT_ref[0, 0:1, :]
    py = predT_ref[0, 1:2, :]
    pz = predT_ref[0, 2:3, :]
    p2row = p2_ref[0]

    rmA = None
    for t in range(_N // _TILE):
        base = t * _TILE
        gx2 = gts_ref[0, pl.ds(base, _TILE), 0:1]
        gy2 = gts_ref[0, pl.ds(base, _TILE), 1:2]
        gz2 = gts_ref[0, pl.ds(base, _TILE), 2:3]
        g2c = g2_ref[0, pl.ds(base, _TILE), :]
        f = g2c + p2row
        f = f + gx2 * px
        f = f + gy2 * py
        f = f + gz2 * pz
        pm = jnp.min(f, axis=0, keepdims=True)
        rmA = pm if rmA is None else jnp.minimum(rmA, pm)
        part = f[:, 0:128]
        for a in range(1, _N // 128):
            part = jnp.minimum(part, f[:, a * 128:(a + 1) * 128])
        s_ref[pl.ds(base, _TILE), :] = part

    sT = jnp.transpose(s_ref[...], (1, 0))
    rmB = jnp.min(sT, axis=0, keepdims=True)
    d1 = jnp.sqrt(jnp.maximum(rmA, 0.0))
    d2 = jnp.sqrt(jnp.maximum(rmB, 0.0))
    loss = _topk_stats(d1, _N, _K) + _topk_stats(d2, _N, _K)
    out_ref[0] = jnp.full((8, 128), loss, jnp.float32)


@jax.jit
def kernel(pred_pointclouds, gt_pointclouds):
    b = pred_pointclouds.shape[0]
    predT = _round_bf16(jnp.transpose(pred_pointclouds, (0, 2, 1)))
    p2 = jnp.sum(pred_pointclouds * pred_pointclouds, axis=-1)[:, None, :]
    gts = -2.0 * _round_bf16(gt_pointclouds)
    g2 = jnp.sum(gt_pointclouds * gt_pointclouds, axis=-1, keepdims=True)
    out = pl.pallas_call(
        _chamfer_body,
        grid=(b,),
        in_specs=[
            pl.BlockSpec((1, 3, _N), lambda i: (i, 0, 0)),
            pl.BlockSpec((1, 1, _N), lambda i: (i, 0, 0)),
            pl.BlockSpec((1, _N, 3), lambda i: (i, 0, 0)),
            pl.BlockSpec((1, _N, 1), lambda i: (i, 0, 0)),
        ],
        out_specs=pl.BlockSpec((1, 8, 128), lambda i: (i, 0, 0)),
        out_shape=jax.ShapeDtypeStruct((b, 8, 128), jnp.float32),
        scratch_shapes=[pltpu.VMEM((_N, 128), jnp.float32)],
        compiler_params=pltpu.CompilerParams(
            dimension_semantics=("parallel",),
        ),
    )(predT, p2, gts, g2)
    return jnp.sum(out[:, 0, 0]) / b


# speedup vs baseline: 2.1031x; 1.3375x over previous
import jax
import jax.numpy as jnp
from jax.experimental import pallas as pl
from jax.experimental.pallas import tpu as pltpu

_N = 4096
_CH = 8
_GRP = 4
_K = _N // 2
_WEIGHT = 3.0


def _round_bf16(x):
    i = jax.lax.bitcast_convert_type(x, jnp.uint32)
    i = (i + jnp.uint32(0x7FFF) + ((i >> 16) & jnp.uint32(1))) & jnp.uint32(0xFFFF0000)
    return jax.lax.bitcast_convert_type(i, jnp.float32)


def _topk_stats(x, n, k):
    mean_all = jnp.sum(x) / n
    xi = jax.lax.bitcast_convert_type(x, jnp.int32)

    def bit_body(b, kth):
        cand = kth | (jnp.int32(1) << (30 - b))
        cnt = jnp.sum(jnp.where(xi >= cand, jnp.int32(1), jnp.int32(0)))
        return jnp.where(cnt >= k, cand, kth)

    kth = jax.lax.fori_loop(0, 31, bit_body, jnp.int32(0))
    thr = jax.lax.bitcast_convert_type(kth, jnp.float32)
    gt_mask = xi > kth
    cnt_gt = jnp.sum(jnp.where(gt_mask, jnp.int32(1), jnp.int32(0)))
    sum_top = jnp.sum(jnp.where(gt_mask, x, 0.0)) + (k - cnt_gt).astype(jnp.float32) * thr
    return mean_all + _WEIGHT * sum_top / k


def _tree_min_cols(f):
    cols = [f[:, a * 128:(a + 1) * 128] for a in range(f.shape[1] // 128)]
    while len(cols) > 1:
        nxt = [jnp.minimum(cols[i], cols[i + 1]) for i in range(0, len(cols) - 1, 2)]
        if len(cols) % 2:
            nxt.append(cols[-1])
        cols = nxt
    return cols[0]


def _chamfer_body(p_ref, gts_ref, g2_ref, out_ref, s_ref, rmA_ref):
    px8 = p_ref[0, 0:_CH, :]
    py8 = p_ref[0, _CH:2 * _CH, :]
    pz8 = p_ref[0, 2 * _CH:3 * _CH, :]
    p28 = p_ref[0, 3 * _CH:4 * _CH, :]

    rmA_ref[...] = jnp.full((_CH, _N), jnp.inf, jnp.float32)

    def it_body(r, carry):
        base = r * (_CH * _GRP)
        acc = None
        for s in range(_GRP):
            rb = base + s * _CH
            gx2 = gts_ref[0, pl.ds(rb, _CH), 0:1]
            gy2 = gts_ref[0, pl.ds(rb, _CH), 1:2]
            gz2 = gts_ref[0, pl.ds(rb, _CH), 2:3]
            g2c = g2_ref[0, pl.ds(rb, _CH), :]
            f = g2c + p28
            f = f + gx2 * px8
            f = f + gy2 * py8
            f = f + gz2 * pz8
            s_ref[pl.ds(rb, _CH), :] = _tree_min_cols(f)
            acc = f if acc is None else jnp.minimum(acc, f)
        rmA_ref[...] = jnp.minimum(rmA_ref[...], acc)
        return carry

    jax.lax.fori_loop(0, _N // (_CH * _GRP), it_body, 0)

    rmA = jnp.min(rmA_ref[...], axis=0, keepdims=True)
    sT = jnp.transpose(s_ref[...], (1, 0))
    rmB = jnp.min(sT, axis=0, keepdims=True)
    d1 = jnp.sqrt(jnp.maximum(rmA, 0.0))
    d2 = jnp.sqrt(jnp.maximum(rmB, 0.0))
    loss = _topk_stats(d1, _N, _K) + _topk_stats(d2, _N, _K)
    out_ref[0] = jnp.full((8, 128), loss, jnp.float32)


@jax.jit
def kernel(pred_pointclouds, gt_pointclouds):
    b = pred_pointclouds.shape[0]
    predT = _round_bf16(jnp.transpose(pred_pointclouds, (0, 2, 1)))
    p2 = jnp.sum(pred_pointclouds * pred_pointclouds, axis=-1)[:, None, :]
    pcat = jnp.concatenate([predT, p2], axis=1)
    p8 = jnp.repeat(pcat, _CH, axis=1)
    gts = -2.0 * _round_bf16(gt_pointclouds)
    g2 = jnp.sum(gt_pointclouds * gt_pointclouds, axis=-1, keepdims=True)
    out = pl.pallas_call(
        _chamfer_body,
        grid=(b,),
        in_specs=[
            pl.BlockSpec((1, 4 * _CH, _N), lambda i: (i, 0, 0)),
            pl.BlockSpec((1, _N, 3), lambda i: (i, 0, 0)),
            pl.BlockSpec((1, _N, 1), lambda i: (i, 0, 0)),
        ],
        out_specs=pl.BlockSpec((1, 8, 128), lambda i: (i, 0, 0)),
        out_shape=jax.ShapeDtypeStruct((b, 8, 128), jnp.float32),
        scratch_shapes=[
            pltpu.VMEM((_N, 128), jnp.float32),
            pltpu.VMEM((_CH, _N), jnp.float32),
        ],
        compiler_params=pltpu.CompilerParams(
            dimension_semantics=("parallel",),
        ),
    )(p8, gts, g2)
    return jnp.sum(out[:, 0, 0]) / b


# speedup vs baseline: 2.8436x; 1.3521x over previous
import jax
import jax.numpy as jnp
from jax.experimental import pallas as pl
from jax.experimental.pallas import tpu as pltpu

_N = 4096
_CH = 8
_GRP = 4
_K = _N // 2
_WEIGHT = 3.0


def _round_bf16(x):
    i = jax.lax.bitcast_convert_type(x, jnp.uint32)
    i = (i + jnp.uint32(0x7FFF) + ((i >> 16) & jnp.uint32(1))) & jnp.uint32(0xFFFF0000)
    return jax.lax.bitcast_convert_type(i, jnp.float32)


def _topk_stats2(x1, x2, n, k):
    mean_all = (jnp.sum(x1) + jnp.sum(x2)) / n
    xi1 = jax.lax.bitcast_convert_type(x1, jnp.int32)
    xi2 = jax.lax.bitcast_convert_type(x2, jnp.int32)

    def bit_body(b, carry):
        k1, k2 = carry
        c1 = k1 | (jnp.int32(1) << (30 - b))
        c2 = k2 | (jnp.int32(1) << (30 - b))
        n1 = jnp.sum(jnp.where(xi1 >= c1, jnp.int32(1), jnp.int32(0)))
        n2 = jnp.sum(jnp.where(xi2 >= c2, jnp.int32(1), jnp.int32(0)))
        return (jnp.where(n1 >= k, c1, k1), jnp.where(n2 >= k, c2, k2))

    k1, k2 = jax.lax.fori_loop(0, 31, bit_body, (jnp.int32(0), jnp.int32(0)))
    t1 = jax.lax.bitcast_convert_type(k1, jnp.float32)
    t2 = jax.lax.bitcast_convert_type(k2, jnp.float32)
    m1 = xi1 > k1
    m2 = xi2 > k2
    c1 = jnp.sum(jnp.where(m1, jnp.int32(1), jnp.int32(0)))
    c2 = jnp.sum(jnp.where(m2, jnp.int32(1), jnp.int32(0)))
    s1 = jnp.sum(jnp.where(m1, x1, 0.0)) + (k - c1).astype(jnp.float32) * t1
    s2 = jnp.sum(jnp.where(m2, x2, 0.0)) + (k - c2).astype(jnp.float32) * t2
    return mean_all + _WEIGHT * (s1 + s2) / k


def _chamfer_body(p_ref, gts_ref, g2_ref, out_ref, s_ref, rmA_ref,
                  gxb_ref, gyb_ref, gzb_ref, g2b_ref):
    gxb_ref[...] = jnp.broadcast_to(gts_ref[0, :, 0:1], (_N, 128))
    gyb_ref[...] = jnp.broadcast_to(gts_ref[0, :, 1:2], (_N, 128))
    gzb_ref[...] = jnp.broadcast_to(gts_ref[0, :, 2:3], (_N, 128))
    g2b_ref[...] = jnp.broadcast_to(g2_ref[0, :, 0:1], (_N, 128))
    rmA_ref[...] = jnp.full((_CH, _N), jnp.inf, jnp.float32)

    def it_body(r, carry):
        base = r * (_CH * _GRP)
        subs = []
        for s in range(_GRP):
            rb = base + s * _CH
            subs.append((
                gxb_ref[pl.ds(rb, _CH), :],
                gyb_ref[pl.ds(rb, _CH), :],
                gzb_ref[pl.ds(rb, _CH), :],
                g2b_ref[pl.ds(rb, _CH), :],
            ))
        partacc = [None] * _GRP
        for L in range(_N // 128):
            sl = slice(L * 128, (L + 1) * 128)
            px = p_ref[0, 0:_CH, sl]
            py = p_ref[0, _CH:2 * _CH, sl]
            pz = p_ref[0, 2 * _CH:3 * _CH, sl]
            p2 = p_ref[0, 3 * _CH:4 * _CH, sl]
            accL = None
            for s, (gxb, gyb, gzb, g2b) in enumerate(subs):
                f = g2b + p2
                f = f + gxb * px
                f = f + gyb * py
                f = f + gzb * pz
                partacc[s] = f if L == 0 else jnp.minimum(partacc[s], f)
                accL = f if accL is None else jnp.minimum(accL, f)
            rmA_ref[:, sl] = jnp.minimum(rmA_ref[:, sl], accL)
        for s in range(_GRP):
            s_ref[pl.ds(base + s * _CH, _CH), :] = partacc[s]
        return carry

    jax.lax.fori_loop(0, _N // (_CH * _GRP), it_body, 0)

    rmA = jnp.min(rmA_ref[...], axis=0, keepdims=True)
    sT = jnp.transpose(s_ref[...], (1, 0))
    rmB = jnp.min(sT, axis=0, keepdims=True)
    d1 = jnp.sqrt(jnp.maximum(rmA, 0.0))
    d2 = jnp.sqrt(jnp.maximum(rmB, 0.0))
    loss = _topk_stats2(d1, d2, _N, _K)
    out_ref[0] = jnp.full((8, 128), loss, jnp.float32)


@jax.jit
def kernel(pred_pointclouds, gt_pointclouds):
    b = pred_pointclouds.shape[0]
    predT = _round_bf16(jnp.transpose(pred_pointclouds, (0, 2, 1)))
    p2 = jnp.sum(pred_pointclouds * pred_pointclouds, axis=-1)[:, None, :]
    pcat = jnp.concatenate([predT, p2], axis=1)
    p8 = jnp.repeat(pcat, _CH, axis=1)
    gts = -2.0 * _round_bf16(gt_pointclouds)
    g2 = jnp.sum(gt_pointclouds * gt_pointclouds, axis=-1, keepdims=True)
    out = pl.pallas_call(
        _chamfer_body,
        grid=(b,),
        in_specs=[
            pl.BlockSpec((1, 4 * _CH, _N), lambda i: (i, 0, 0)),
            pl.BlockSpec((1, _N, 3), lambda i: (i, 0, 0)),
            pl.BlockSpec((1, _N, 1), lambda i: (i, 0, 0)),
        ],
        out_specs=pl.BlockSpec((1, 8, 128), lambda i: (i, 0, 0)),
        out_shape=jax.ShapeDtypeStruct((b, 8, 128), jnp.float32),
        scratch_shapes=[
            pltpu.VMEM((_N, 128), jnp.float32),
            pltpu.VMEM((_CH, _N), jnp.float32),
            pltpu.VMEM((_N, 128), jnp.float32),
            pltpu.VMEM((_N, 128), jnp.float32),
            pltpu.VMEM((_N, 128), jnp.float32),
            pltpu.VMEM((_N, 128), jnp.float32),
        ],
        compiler_params=pltpu.CompilerParams(
            dimension_semantics=("parallel",),
        ),
    )(p8, gts, g2)
    return jnp.sum(out[:, 0, 0]) / b


# speedup vs baseline: 2.8636x; 1.0070x over previous
import jax
import jax.numpy as jnp
from jax.experimental import pallas as pl
from jax.experimental.pallas import tpu as pltpu

_N = 4096
_CH = 8
_GRP = 8
_K = _N // 2
_WEIGHT = 3.0


def _round_bf16(x):
    i = jax.lax.bitcast_convert_type(x, jnp.uint32)
    i = (i + jnp.uint32(0x7FFF) + ((i >> 16) & jnp.uint32(1))) & jnp.uint32(0xFFFF0000)
    return jax.lax.bitcast_convert_type(i, jnp.float32)


def _topk_stats2(x1, x2, n, k):
    mean_all = (jnp.sum(x1) + jnp.sum(x2)) / n
    xi1 = jax.lax.bitcast_convert_type(x1, jnp.int32)
    xi2 = jax.lax.bitcast_convert_type(x2, jnp.int32)

    def bit_body(b, carry):
        k1, k2 = carry
        c1 = k1 | (jnp.int32(1) << (30 - b))
        c2 = k2 | (jnp.int32(1) << (30 - b))
        n1 = jnp.sum(jnp.where(xi1 >= c1, jnp.int32(1), jnp.int32(0)))
        n2 = jnp.sum(jnp.where(xi2 >= c2, jnp.int32(1), jnp.int32(0)))
        return (jnp.where(n1 >= k, c1, k1), jnp.where(n2 >= k, c2, k2))

    k1, k2 = jax.lax.fori_loop(0, 31, bit_body, (jnp.int32(0), jnp.int32(0)))
    t1 = jax.lax.bitcast_convert_type(k1, jnp.float32)
    t2 = jax.lax.bitcast_convert_type(k2, jnp.float32)
    m1 = xi1 > k1
    m2 = xi2 > k2
    c1 = jnp.sum(jnp.where(m1, jnp.int32(1), jnp.int32(0)))
    c2 = jnp.sum(jnp.where(m2, jnp.int32(1), jnp.int32(0)))
    s1 = jnp.sum(jnp.where(m1, x1, 0.0)) + (k - c1).astype(jnp.float32) * t1
    s2 = jnp.sum(jnp.where(m2, x2, 0.0)) + (k - c2).astype(jnp.float32) * t2
    return mean_all + _WEIGHT * (s1 + s2) / k


def _chamfer_body(p_ref, gts_ref, g2_ref, out_ref, s_ref, rmA_ref,
                  gxb_ref, gyb_ref, gzb_ref, g2b_ref):
    gxb_ref[...] = jnp.broadcast_to(gts_ref[0, :, 0:1], (_N, 128))
    gyb_ref[...] = jnp.broadcast_to(gts_ref[0, :, 1:2], (_N, 128))
    gzb_ref[...] = jnp.broadcast_to(gts_ref[0, :, 2:3], (_N, 128))
    g2b_ref[...] = jnp.broadcast_to(g2_ref[0, :, 0:1], (_N, 128))
    rmA_ref[...] = jnp.full((_CH, _N), jnp.inf, jnp.float32)

    def it_body(r, carry):
        base = r * (_CH * _GRP)
        subs = []
        for s in range(_GRP):
            rb = base + s * _CH
            subs.append((
                gxb_ref[pl.ds(rb, _CH), :],
                gyb_ref[pl.ds(rb, _CH), :],
                gzb_ref[pl.ds(rb, _CH), :],
                g2b_ref[pl.ds(rb, _CH), :],
            ))
        partacc = [None] * _GRP
        for L in range(_N // 128):
            sl = slice(L * 128, (L + 1) * 128)
            px = p_ref[0, 0:_CH, sl]
            py = p_ref[0, _CH:2 * _CH, sl]
            pz = p_ref[0, 2 * _CH:3 * _CH, sl]
            p2 = p_ref[0, 3 * _CH:4 * _CH, sl]
            accL = None
            for s, (gxb, gyb, gzb, g2b) in enumerate(subs):
                f = g2b + p2
                f = f + gxb * px
                f = f + gyb * py
                f = f + gzb * pz
                partacc[s] = f if L == 0 else jnp.minimum(partacc[s], f)
                accL = f if accL is None else jnp.minimum(accL, f)
            rmA_ref[:, sl] = jnp.minimum(rmA_ref[:, sl], accL)
        for s in range(_GRP):
            s_ref[pl.ds(base + s * _CH, _CH), :] = partacc[s]
        return carry

    jax.lax.fori_loop(0, _N // (_CH * _GRP), it_body, 0)

    rmA = jnp.min(rmA_ref[...], axis=0, keepdims=True)
    sT = jnp.transpose(s_ref[...], (1, 0))
    rmB = jnp.min(sT, axis=0, keepdims=True)
    d1 = jnp.sqrt(jnp.maximum(rmA, 0.0))
    d2 = jnp.sqrt(jnp.maximum(rmB, 0.0))
    loss = _topk_stats2(d1, d2, _N, _K)
    out_ref[0] = jnp.full((8, 128), loss, jnp.float32)


@jax.jit
def kernel(pred_pointclouds, gt_pointclouds):
    b = pred_pointclouds.shape[0]
    predT = _round_bf16(jnp.transpose(pred_pointclouds, (0, 2, 1)))
    p2 = jnp.sum(pred_pointclouds * pred_pointclouds, axis=-1)[:, None, :]
    pcat = jnp.concatenate([predT, p2], axis=1)
    p8 = jnp.repeat(pcat, _CH, axis=1)
    gts = -2.0 * _round_bf16(gt_pointclouds)
    g2 = jnp.sum(gt_pointclouds * gt_pointclouds, axis=-1, keepdims=True)
    out = pl.pallas_call(
        _chamfer_body,
        grid=(b,),
        in_specs=[
            pl.BlockSpec((1, 4 * _CH, _N), lambda i: (i, 0, 0)),
            pl.BlockSpec((1, _N, 3), lambda i: (i, 0, 0)),
            pl.BlockSpec((1, _N, 1), lambda i: (i, 0, 0)),
        ],
        out_specs=pl.BlockSpec((1, 8, 128), lambda i: (i, 0, 0)),
        out_shape=jax.ShapeDtypeStruct((b, 8, 128), jnp.float32),
        scratch_shapes=[
            pltpu.VMEM((_N, 128), jnp.float32),
            pltpu.VMEM((_CH, _N), jnp.float32),
            pltpu.VMEM((_N, 128), jnp.float32),
            pltpu.VMEM((_N, 128), jnp.float32),
            pltpu.VMEM((_N, 128), jnp.float32),
            pltpu.VMEM((_N, 128), jnp.float32),
        ],
        compiler_params=pltpu.CompilerParams(
            dimension_semantics=("parallel",),
        ),
    )(p8, gts, g2)
    return jnp.sum(out[:, 0, 0]) / b


# speedup vs baseline: 2.9001x; 1.0127x over previous
import jax
import jax.numpy as jnp
from jax.experimental import pallas as pl
from jax.experimental.pallas import tpu as pltpu

_N = 4096
_CH = 8
_GRP = 8
_K = _N // 2
_WEIGHT = 3.0


def _round_bf16(x):
    i = jax.lax.bitcast_convert_type(x, jnp.uint32)
    i = (i + jnp.uint32(0x7FFF) + ((i >> 16) & jnp.uint32(1))) & jnp.uint32(0xFFFF0000)
    return jax.lax.bitcast_convert_type(i, jnp.float32)


def _topk_stats2(x1, x2, n, k):
    mean_all = (jnp.sum(x1) + jnp.sum(x2)) / n
    xi1 = jax.lax.bitcast_convert_type(x1, jnp.int32)
    xi2 = jax.lax.bitcast_convert_type(x2, jnp.int32)

    def bit_body(b, carry):
        k1, k2 = carry
        c1 = k1 | (jnp.int32(1) << (30 - b))
        c2 = k2 | (jnp.int32(1) << (30 - b))
        n1 = jnp.sum(jnp.where(xi1 >= c1, jnp.int32(1), jnp.int32(0)))
        n2 = jnp.sum(jnp.where(xi2 >= c2, jnp.int32(1), jnp.int32(0)))
        return (jnp.where(n1 >= k, c1, k1), jnp.where(n2 >= k, c2, k2))

    k1, k2 = jax.lax.fori_loop(0, 31, bit_body, (jnp.int32(0), jnp.int32(0)))
    t1 = jax.lax.bitcast_convert_type(k1, jnp.float32)
    t2 = jax.lax.bitcast_convert_type(k2, jnp.float32)
    m1 = xi1 > k1
    m2 = xi2 > k2
    c1 = jnp.sum(jnp.where(m1, jnp.int32(1), jnp.int32(0)))
    c2 = jnp.sum(jnp.where(m2, jnp.int32(1), jnp.int32(0)))
    s1 = jnp.sum(jnp.where(m1, x1, 0.0)) + (k - c1).astype(jnp.float32) * t1
    s2 = jnp.sum(jnp.where(m2, x2, 0.0)) + (k - c2).astype(jnp.float32) * t2
    return mean_all + _WEIGHT * (s1 + s2) / k


def _chamfer_body(p_ref, gts_ref, g2_ref, out_ref, s_ref, rmA_ref,
                  gxb_ref, gyb_ref, gzb_ref, g2b_ref):
    gxb_ref[...] = jnp.broadcast_to(gts_ref[0, :, 0:1], (_N, 128))
    gyb_ref[...] = jnp.broadcast_to(gts_ref[0, :, 1:2], (_N, 128))
    gzb_ref[...] = jnp.broadcast_to(gts_ref[0, :, 2:3], (_N, 128))
    g2b_ref[...] = jnp.broadcast_to(g2_ref[0, :, 0:1], (_N, 128))
    rmA_ref[...] = jnp.full((_CH, _N), jnp.inf, jnp.float32)

    def it_body(r, carry):
        base = r * (_CH * _GRP)
        subs = []
        for s in range(_GRP):
            rb = base + s * _CH
            subs.append((
                gxb_ref[pl.ds(rb, _CH), :],
                gyb_ref[pl.ds(rb, _CH), :],
                gzb_ref[pl.ds(rb, _CH), :],
                g2b_ref[pl.ds(rb, _CH), :],
            ))
        partacc = [None] * _GRP
        for L in range(_N // 128):
            sl = slice(L * 128, (L + 1) * 128)
            px = p_ref[0, 0:_CH, sl]
            py = p_ref[0, _CH:2 * _CH, sl]
            pz = p_ref[0, 2 * _CH:3 * _CH, sl]
            p2 = p_ref[0, 3 * _CH:4 * _CH, sl]
            accL = None
            for s, (gxb, gyb, gzb, g2b) in enumerate(subs):
                f = g2b + p2
                f = f + gxb * px
                f = f + gyb * py
                f = f + gzb * pz
                partacc[s] = f if L == 0 else jnp.minimum(partacc[s], f)
                accL = f if accL is None else jnp.minimum(accL, f)
            rmA_ref[:, sl] = jnp.minimum(rmA_ref[:, sl], accL)
        for s in range(_GRP):
            s_ref[pl.ds(base + s * _CH, _CH), :] = partacc[s]
        return carry

    jax.lax.fori_loop(0, _N // (_CH * _GRP), it_body, 0)

    rmA = jnp.min(rmA_ref[...], axis=0, keepdims=True)
    sT = jnp.transpose(s_ref[...], (1, 0))
    rmB = jnp.min(sT, axis=0, keepdims=True)
    d1 = jnp.sqrt(jnp.maximum(rmA, 0.0))
    d2 = jnp.sqrt(jnp.maximum(rmB, 0.0))
    loss = _topk_stats2(d1.reshape(8, 512), d2.reshape(8, 512), _N, _K)
    out_ref[0] = jnp.full((8, 128), loss, jnp.float32)


@jax.jit
def kernel(pred_pointclouds, gt_pointclouds):
    b = pred_pointclouds.shape[0]
    predT = _round_bf16(jnp.transpose(pred_pointclouds, (0, 2, 1)))
    p2 = jnp.sum(pred_pointclouds * pred_pointclouds, axis=-1)[:, None, :]
    pcat = jnp.concatenate([predT, p2], axis=1)
    p8 = jnp.repeat(pcat, _CH, axis=1)
    gts = -2.0 * _round_bf16(gt_pointclouds)
    g2 = jnp.sum(gt_pointclouds * gt_pointclouds, axis=-1, keepdims=True)
    out = pl.pallas_call(
        _chamfer_body,
        grid=(b,),
        in_specs=[
            pl.BlockSpec((1, 4 * _CH, _N), lambda i: (i, 0, 0)),
            pl.BlockSpec((1, _N, 3), lambda i: (i, 0, 0)),
            pl.BlockSpec((1, _N, 1), lambda i: (i, 0, 0)),
        ],
        out_specs=pl.BlockSpec((1, 8, 128), lambda i: (i, 0, 0)),
        out_shape=jax.ShapeDtypeStruct((b, 8, 128), jnp.float32),
        scratch_shapes=[
            pltpu.VMEM((_N, 128), jnp.float32),
            pltpu.VMEM((_CH, _N), jnp.float32),
            pltpu.VMEM((_N, 128), jnp.float32),
            pltpu.VMEM((_N, 128), jnp.float32),
            pltpu.VMEM((_N, 128), jnp.float32),
            pltpu.VMEM((_N, 128), jnp.float32),
        ],
        compiler_params=pltpu.CompilerParams(
            dimension_semantics=("parallel",),
        ),
    )(p8, gts, g2)
    return jnp.sum(out[:, 0, 0]) / b
